# Initial kernel scaffold; baseline (speedup 1.0000x reference)
import jax
import jax.numpy as jnp
from jax.experimental import pallas as pl


def kernel(x, edge_src, edge_dst, node_batch, W_enc, b_enc, enc_W1, enc_b1, enc_W2, enc_b2, Wa1, ba1, Wa2, ba2, W_cenc, b_cenc, clf_W1, clf_b1, clf_W2, clf_b2, W_out, b_out):
    raise NotImplementedError("write your pallas kernel here")


# speedup vs baseline: 51.7438x; 51.7438x over previous
import functools
import jax
import jax.numpy as jnp
from jax.experimental import pallas as pl
from jax.experimental.pallas import tpu as pltpu

N = 10000
E = 160000
B = 64
DF = 128
EMB = 128
NL = 3
OUT = 10
RATIO = 0.25
MAX_ITER = 30
EPS = 1e-20

ER = 1250
LANE = 128
NEG_INF = -jnp.inf


def _sinkhorn_body(att_ref, eb_ref, stats_ref, t_ref):
    att = att_ref[...]
    eb = eb_ref[...]
    gsum = stats_ref[0, :B]
    gsq = stats_ref[1, :B]
    gmin = stats_ref[2, :B]
    degf = stats_ref[3, :B]

    mean = gsum / degf
    var = (gsq - degf * mean * mean) / (degf - 1.0)
    std = jnp.sqrt(var)
    pmin = (gmin - mean) / std
    lmax = jnp.max(degf)

    meanpe = jnp.zeros_like(att)
    stdpe = jnp.zeros_like(att)
    for b in range(B):
        cmp = eb == b
        meanpe = meanpe + jnp.where(cmp, mean[b], 0.0)
        stdpe = stdpe + jnp.where(cmp, std[b], 0.0)
    attn = (att - meanpe) / stdpe

    m = lmax - degf
    logm = jnp.where(m > 0, jnp.log(jnp.maximum(m, 1e-30)), NEG_INF)
    log_rs0 = jnp.log(lmax - RATIO * degf + EPS)
    log_rs1 = jnp.log(RATIO * degf + EPS)

    y0 = -attn
    y1 = attn - 1.0
    q0 = -pmin
    q1 = pmin - 1.0

    def body(_, carry):
        y0, y1, q0, q1 = carry
        mx = jnp.maximum(y0, y1)
        lse = mx + jnp.log(jnp.exp(y0 - mx) + jnp.exp(y1 - mx))
        y0 = y0 - lse
        y1 = y1 - lse
        qm = jnp.maximum(q0, q1)
        qlse = qm + jnp.log(jnp.exp(q0 - qm) + jnp.exp(q1 - qm))
        q0 = q0 - qlse
        q1 = q1 - qlse
        e0 = jnp.exp(y0)
        e1 = jnp.exp(y1)
        se0 = []
        se1 = []
        for b in range(B):
            cmp = eb == b
            se0.append(jnp.sum(jnp.where(cmp, e0, 0.0)))
            se1.append(jnp.sum(jnp.where(cmp, e1, 0.0)))
        se0 = jnp.stack(se0) + jnp.exp(q0 + logm)
        se1 = jnp.stack(se1) + jnp.exp(q1 + logm)
        c0 = log_rs0 - jnp.log(se0)
        c1 = log_rs1 - jnp.log(se1)
        q0 = q0 + c0
        q1 = q1 + c1
        d0 = jnp.zeros_like(y0)
        d1 = jnp.zeros_like(y1)
        for b in range(B):
            cmp = eb == b
            d0 = d0 + jnp.where(cmp, c0[b], 0.0)
            d1 = d1 + jnp.where(cmp, c1[b], 0.0)
        return (y0 + d0, y1 + d1, q0, q1)

    y0, y1, q0, q1 = jax.lax.fori_loop(0, MAX_ITER, body, (y0, y1, q0, q1))
    t_ref[...] = jnp.exp(y1)


def _sinkhorn(att2d, eb2d, stats, interpret=False):
    return pl.pallas_call(
        _sinkhorn_body,
        out_shape=jax.ShapeDtypeStruct((ER, LANE), jnp.float32),
        interpret=interpret,
    )(att2d, eb2d, stats)


def _gin(x, src, dst, Wenc, benc, W1, b1, W2, b2, edge_att=None):
    h = x @ Wenc + benc
    nl = W1.shape[0]
    for l in range(nl):
        msg = h[dst]
        if edge_att is not None:
            msg = msg * edge_att[:, None]
        agg = jnp.zeros_like(h).at[src].add(msg)
        z = h + agg
        z = jax.nn.relu(z @ W1[l] + b1[l])
        z = z @ W2[l] + b2[l]
        h = jax.nn.relu(z) if l < nl - 1 else z
    return h


@jax.jit
def kernel(x, edge_src, edge_dst, node_batch, W_enc, b_enc, enc_W1, enc_b1,
           enc_W2, enc_b2, Wa1, ba1, Wa2, ba2, W_cenc, b_cenc, clf_W1, clf_b1,
           clf_W2, clf_b2, W_out, b_out):
    edge_batch = node_batch[edge_src]
    deg = jnp.bincount(edge_batch, length=B)
    degf = deg.astype(jnp.float32)

    h = _gin(x, edge_src, edge_dst, W_enc, b_enc, enc_W1, enc_b1, enc_W2, enc_b2)
    att = (jax.nn.relu(jnp.concatenate([h[edge_src], h[edge_dst]], -1) @ Wa1 + ba1) @ Wa2 + ba2)[:, 0]

    gsum = jax.ops.segment_sum(att, edge_batch, num_segments=B)
    gsq = jax.ops.segment_sum(att * att, edge_batch, num_segments=B)
    gmin = jax.ops.segment_min(att, edge_batch, num_segments=B)
    stats = jnp.zeros((8, LANE), jnp.float32)
    stats = stats.at[0, :B].set(gsum).at[1, :B].set(gsq)
    stats = stats.at[2, :B].set(gmin).at[3, :B].set(degf)

    T2d = _sinkhorn(att.reshape(ER, LANE), edge_batch.reshape(ER, LANE), stats)
    T_edge = T2d.reshape(E)

    hc = _gin(x, edge_src, edge_dst, W_cenc, b_cenc, clf_W1, clf_b1, clf_W2, clf_b2, edge_att=T_edge)
    ncount = jnp.bincount(node_batch, length=B).astype(jnp.float32)
    hg = jax.ops.segment_sum(hc, node_batch, num_segments=B) / ncount[:, None]
    return hg @ W_out + b_out


# speedup vs baseline: 78.2057x; 1.5114x over previous
import functools
import jax
import jax.numpy as jnp
from jax import lax
from jax.experimental import pallas as pl
from jax.experimental.pallas import tpu as pltpu
from jax.experimental.pallas import tpu_sc as plsc

N = 10000
E = 160000
B = 64
DF = 128
EMB = 128
NL = 3
OUT = 10
RATIO = 0.25
MAX_ITER = 30
EPS = 1e-20

ER = 1250
LANE = 128
NEG_INF = -jnp.inf

NW = 32
EPADE = 163840
PERW = EPADE // NW
CHUNK = 128
NCHK = PERW // CHUNK
NP = 10240
ROWS_PER_SUB = NP // 16


def _make_sc_agg(weighted):
    mesh = plsc.VectorSubcoreMesh(core_axis_name="c", subcore_axis_name="s")
    scratch = [
        pltpu.VMEM((CHUNK,), jnp.int32),
        pltpu.VMEM((CHUNK,), jnp.int32),
        pltpu.VMEM((CHUNK + 16,), jnp.float32),
        pltpu.VMEM((CHUNK, EMB), jnp.float32),
        pltpu.VMEM((CHUNK, EMB), jnp.float32),
        pltpu.VMEM_SHARED((NP, EMB), jnp.float32),
        pltpu.SemaphoreType.DMA,
    ]

    @functools.partial(
        pl.kernel,
        out_type=jax.ShapeDtypeStruct((2, NP, EMB), jnp.float32),
        mesh=mesh,
        scratch_types=scratch,
    )
    def k(h_hbm, dst_hbm, src_hbm, w_hbm, out_hbm, dsti, srci, wv, rows, zbuf,
          acc, sem):
        cid = lax.axis_index("c")
        sid = lax.axis_index("s")
        wid = cid * 16 + sid

        def zrow(r, _):
            for j in range(8):
                zbuf[r, 16 * j:16 * (j + 1)] = jnp.zeros((16,), jnp.float32)
            return 0

        lax.fori_loop(0, CHUNK, zrow, 0)
        for z in range(ROWS_PER_SUB // CHUNK):
            pltpu.sync_copy(zbuf, acc.at[pl.ds(sid * ROWS_PER_SUB + z * CHUNK, CHUNK)])
        plsc.subcore_barrier()

        base = wid * PERW

        def chunkfn(ci, _):
            off = base + ci * CHUNK
            pltpu.sync_copy(dst_hbm.at[pl.ds(off, CHUNK)], dsti)
            pltpu.sync_copy(src_hbm.at[pl.ds(off, CHUNK)], srci)
            pltpu.async_copy(h_hbm.at[dsti], rows, sem).wait()
            if weighted:
                pltpu.sync_copy(w_hbm.at[pl.ds(off, CHUNK)], wv.at[pl.ds(0, CHUNK)])

                def mrow(r, _):
                    wr = wv[pl.ds(r, 16)][0]
                    for j in range(8):
                        sl = pl.ds(16 * j, 16)
                        rows[r, sl] = rows[r, sl] * wr
                    return 0

                lax.fori_loop(0, CHUNK, mrow, 0)
            pltpu.sync_copy(rows, acc.at[srci], add=True)
            return 0

        lax.fori_loop(0, NCHK, chunkfn, 0)
        plsc.subcore_barrier()
        for z in range(ROWS_PER_SUB // CHUNK):
            sl = pl.ds(sid * ROWS_PER_SUB + z * CHUNK, CHUNK)
            pltpu.sync_copy(acc.at[sl], out_hbm.at[cid, sl])

    return k


_sc_agg_w = _make_sc_agg(True)
_sc_agg_nw = _make_sc_agg(False)


def _sc_agg(h_np, dstp, srcp, w=None):
    if w is None:
        parts = _sc_agg_nw(h_np, dstp, srcp, jnp.zeros((EPADE,), jnp.float32))
    else:
        parts = _sc_agg_w(h_np, dstp, srcp, w)
    return parts[0] + parts[1]


def _sinkhorn_body(att_ref, eb_ref, stats_ref, t_ref):
    att = att_ref[...]
    eb = eb_ref[...]
    gsum = stats_ref[0, :B]
    gsq = stats_ref[1, :B]
    gmin = stats_ref[2, :B]
    degf = stats_ref[3, :B]

    mean = gsum / degf
    var = (gsq - degf * mean * mean) / (degf - 1.0)
    std = jnp.sqrt(var)
    pmin = (gmin - mean) / std
    lmax = jnp.max(degf)

    meanpe = jnp.zeros_like(att)
    stdpe = jnp.zeros_like(att)
    for b in range(B):
        cmp = eb == b
        meanpe = meanpe + jnp.where(cmp, mean[b], 0.0)
        stdpe = stdpe + jnp.where(cmp, std[b], 0.0)
    attn = (att - meanpe) / stdpe

    m = lmax - degf
    logm = jnp.where(m > 0, jnp.log(jnp.maximum(m, 1e-30)), NEG_INF)
    log_rs0 = jnp.log(lmax - RATIO * degf + EPS)
    log_rs1 = jnp.log(RATIO * degf + EPS)

    y0 = -attn
    y1 = attn - 1.0
    q0 = -pmin
    q1 = pmin - 1.0

    def body(_, carry):
        y0, y1, q0, q1 = carry
        mx = jnp.maximum(y0, y1)
        lse = mx + jnp.log(jnp.exp(y0 - mx) + jnp.exp(y1 - mx))
        y0 = y0 - lse
        y1 = y1 - lse
        qm = jnp.maximum(q0, q1)
        qlse = qm + jnp.log(jnp.exp(q0 - qm) + jnp.exp(q1 - qm))
        q0 = q0 - qlse
        q1 = q1 - qlse
        e0 = jnp.exp(y0)
        e1 = jnp.exp(y1)
        se0 = []
        se1 = []
        for b in range(B):
            cmp = eb == b
            se0.append(jnp.sum(jnp.where(cmp, e0, 0.0)))
            se1.append(jnp.sum(jnp.where(cmp, e1, 0.0)))
        se0 = jnp.stack(se0) + jnp.exp(q0 + logm)
        se1 = jnp.stack(se1) + jnp.exp(q1 + logm)
        c0 = log_rs0 - jnp.log(se0)
        c1 = log_rs1 - jnp.log(se1)
        q0 = q0 + c0
        q1 = q1 + c1
        d0 = jnp.zeros_like(y0)
        d1 = jnp.zeros_like(y1)
        for b in range(B):
            cmp = eb == b
            d0 = d0 + jnp.where(cmp, c0[b], 0.0)
            d1 = d1 + jnp.where(cmp, c1[b], 0.0)
        return (y0 + d0, y1 + d1, q0, q1)

    y0, y1, q0, q1 = jax.lax.fori_loop(0, MAX_ITER, body, (y0, y1, q0, q1))
    t_ref[...] = jnp.exp(y1)


def _sinkhorn(att2d, eb2d, stats, interpret=False):
    return pl.pallas_call(
        _sinkhorn_body,
        out_shape=jax.ShapeDtypeStruct((ER, LANE), jnp.float32),
        interpret=interpret,
    )(att2d, eb2d, stats)


def _gin(x_np, srcp, dstp, Wenc, benc, W1, b1, W2, b2, edge_att=None):
    h = x_np @ Wenc + benc
    nl = W1.shape[0]
    for l in range(nl):
        agg = _sc_agg(h, dstp, srcp, edge_att)
        z = h + agg
        z = jax.nn.relu(z @ W1[l] + b1[l])
        z = z @ W2[l] + b2[l]
        h = jax.nn.relu(z) if l < nl - 1 else z
    return h


@jax.jit
def kernel(x, edge_src, edge_dst, node_batch, W_enc, b_enc, enc_W1, enc_b1,
           enc_W2, enc_b2, Wa1, ba1, Wa2, ba2, W_cenc, b_cenc, clf_W1, clf_b1,
           clf_W2, clf_b2, W_out, b_out):
    edge_batch = node_batch[edge_src]
    deg = jnp.bincount(edge_batch, length=B)
    degf = deg.astype(jnp.float32)

    x_np = jnp.pad(x, ((0, NP - N), (0, 0)))
    srcp = jnp.pad(edge_src, (0, EPADE - E), constant_values=N)
    dstp = jnp.pad(edge_dst, (0, EPADE - E), constant_values=0)

    h = _gin(x_np, srcp, dstp, W_enc, b_enc, enc_W1, enc_b1, enc_W2, enc_b2)
    hs = h[edge_src]
    hd = h[edge_dst]
    att = (jax.nn.relu(jnp.concatenate([hs, hd], -1) @ Wa1 + ba1) @ Wa2 + ba2)[:, 0]

    gsum = jax.ops.segment_sum(att, edge_batch, num_segments=B)
    gsq = jax.ops.segment_sum(att * att, edge_batch, num_segments=B)
    gmin = jax.ops.segment_min(att, edge_batch, num_segments=B)
    stats = jnp.zeros((8, LANE), jnp.float32)
    stats = stats.at[0, :B].set(gsum).at[1, :B].set(gsq)
    stats = stats.at[2, :B].set(gmin).at[3, :B].set(degf)

    T2d = _sinkhorn(att.reshape(ER, LANE), edge_batch.reshape(ER, LANE), stats)
    T_edge = T2d.reshape(E)

    Tp = jnp.pad(T_edge, (0, EPADE - E))
    hc = _gin(x_np, srcp, dstp, W_cenc, b_cenc, clf_W1, clf_b1, clf_W2,
              clf_b2, edge_att=Tp)[:N]
    ncount = jnp.bincount(node_batch, length=B).astype(jnp.float32)
    hg = jax.ops.segment_sum(hc, node_batch, num_segments=B) / ncount[:, None]
    return hg @ W_out + b_out


# speedup vs baseline: 82.6596x; 1.0570x over previous
import functools
import jax
import jax.numpy as jnp
from jax import lax
from jax.experimental import pallas as pl
from jax.experimental.pallas import tpu as pltpu
from jax.experimental.pallas import tpu_sc as plsc

N = 10000
E = 160000
B = 64
DF = 128
EMB = 128
NL = 3
OUT = 10
RATIO = 0.25
MAX_ITER = 30
EPS = 1e-20

ER = 1250
LANE = 128
NEG_INF = -jnp.inf

NW = 32
EPADE = 163840
PERW = EPADE // NW
CHUNK = 128
NCHK = PERW // CHUNK
NP = 10240
ROWS_PER_SUB = NP // 16


NBUF = 2
NGRP = NCHK // NBUF


def _make_sc_agg(weighted):
    mesh = plsc.VectorSubcoreMesh(core_axis_name="c", subcore_axis_name="s")
    scratch = [
        pltpu.VMEM((PERW,), jnp.int32),
        pltpu.VMEM((PERW,), jnp.int32),
        pltpu.VMEM((PERW + 16,), jnp.float32),
    ] + [pltpu.VMEM((CHUNK, EMB), jnp.float32) for _ in range(NBUF)] + [
        pltpu.VMEM_SHARED((NP, EMB), jnp.float32),
    ] + [pltpu.SemaphoreType.DMA for _ in range(NBUF)]

    @functools.partial(
        pl.kernel,
        out_type=jax.ShapeDtypeStruct((2, NP, EMB), jnp.float32),
        mesh=mesh,
        scratch_types=scratch,
    )
    def k(h_hbm, dst_hbm, src_hbm, w_hbm, out_hbm, dsts, srcs, ws,
          r0, r1, acc, s0, s1):
        rows = [r0, r1]
        sems = [s0, s1]
        cid = lax.axis_index("c")
        sid = lax.axis_index("s")
        wid = cid * 16 + sid

        def zrow(r, _):
            for j in range(8):
                r0[r, 16 * j:16 * (j + 1)] = jnp.zeros((16,), jnp.float32)
            return 0

        lax.fori_loop(0, CHUNK, zrow, 0)
        for z in range(ROWS_PER_SUB // CHUNK):
            pltpu.sync_copy(r0, acc.at[pl.ds(sid * ROWS_PER_SUB + z * CHUNK, CHUNK)])
        plsc.subcore_barrier()

        base = wid * PERW
        pltpu.sync_copy(dst_hbm.at[pl.ds(base, PERW)], dsts)
        pltpu.sync_copy(src_hbm.at[pl.ds(base, PERW)], srcs)
        if weighted:
            pltpu.sync_copy(w_hbm.at[pl.ds(base, PERW)], ws.at[pl.ds(0, PERW)])

        def grpfn(g, _):
            goff = g * (NBUF * CHUNK)
            handles = []
            for b in range(NBUF):
                off = goff + b * CHUNK
                h = pltpu.async_copy(
                    h_hbm.at[dsts.at[pl.ds(off, CHUNK)]], rows[b], sems[b])
                handles.append((h, off))
            for b in range(NBUF):
                h, off = handles[b]
                h.wait()
                if weighted:
                    def mrow(r, _, b=b, off=off):
                        wr = ws[pl.ds(off + r, 16)][0]
                        for j in range(8):
                            sl = pl.ds(16 * j, 16)
                            rows[b][r, sl] = rows[b][r, sl] * wr
                        return 0

                    lax.fori_loop(0, CHUNK, mrow, 0)
                pltpu.sync_copy(rows[b], acc.at[srcs.at[pl.ds(off, CHUNK)]],
                                add=True)
            return 0

        lax.fori_loop(0, NGRP, grpfn, 0)
        plsc.subcore_barrier()
        for z in range(ROWS_PER_SUB // CHUNK):
            sl = pl.ds(sid * ROWS_PER_SUB + z * CHUNK, CHUNK)
            pltpu.sync_copy(acc.at[sl], out_hbm.at[cid, sl])

    return k


_sc_agg_w = _make_sc_agg(True)
_sc_agg_nw = _make_sc_agg(False)


def _sc_agg(h_np, dstp, srcp, w=None):
    if w is None:
        parts = _sc_agg_nw(h_np, dstp, srcp, jnp.zeros((EPADE,), jnp.float32))
    else:
        parts = _sc_agg_w(h_np, dstp, srcp, w)
    return parts[0] + parts[1]


def _sinkhorn_body(att_ref, eb_ref, stats_ref, t_ref):
    att = att_ref[...]
    eb = eb_ref[...]
    gsum = stats_ref[0, :B]
    gsq = stats_ref[1, :B]
    gmin = stats_ref[2, :B]
    degf = stats_ref[3, :B]

    mean = gsum / degf
    var = (gsq - degf * mean * mean) / (degf - 1.0)
    std = jnp.sqrt(var)
    pmin = (gmin - mean) / std
    lmax = jnp.max(degf)

    meanpe = jnp.zeros_like(att)
    stdpe = jnp.zeros_like(att)
    for b in range(B):
        cmp = eb == b
        meanpe = meanpe + jnp.where(cmp, mean[b], 0.0)
        stdpe = stdpe + jnp.where(cmp, std[b], 0.0)
    attn = (att - meanpe) / stdpe

    m = lmax - degf
    logm = jnp.where(m > 0, jnp.log(jnp.maximum(m, 1e-30)), NEG_INF)
    log_rs0 = jnp.log(lmax - RATIO * degf + EPS)
    log_rs1 = jnp.log(RATIO * degf + EPS)

    y0 = -attn
    y1 = attn - 1.0
    q0 = -pmin
    q1 = pmin - 1.0

    def body(_, carry):
        y0, y1, q0, q1 = carry
        mx = jnp.maximum(y0, y1)
        lse = mx + jnp.log(jnp.exp(y0 - mx) + jnp.exp(y1 - mx))
        y0 = y0 - lse
        y1 = y1 - lse
        qm = jnp.maximum(q0, q1)
        qlse = qm + jnp.log(jnp.exp(q0 - qm) + jnp.exp(q1 - qm))
        q0 = q0 - qlse
        q1 = q1 - qlse
        e0 = jnp.exp(y0)
        e1 = jnp.exp(y1)
        se0 = []
        se1 = []
        for b in range(B):
            cmp = eb == b
            se0.append(jnp.sum(jnp.where(cmp, e0, 0.0)))
            se1.append(jnp.sum(jnp.where(cmp, e1, 0.0)))
        se0 = jnp.stack(se0) + jnp.exp(q0 + logm)
        se1 = jnp.stack(se1) + jnp.exp(q1 + logm)
        c0 = log_rs0 - jnp.log(se0)
        c1 = log_rs1 - jnp.log(se1)
        q0 = q0 + c0
        q1 = q1 + c1
        d0 = jnp.zeros_like(y0)
        d1 = jnp.zeros_like(y1)
        for b in range(B):
            cmp = eb == b
            d0 = d0 + jnp.where(cmp, c0[b], 0.0)
            d1 = d1 + jnp.where(cmp, c1[b], 0.0)
        return (y0 + d0, y1 + d1, q0, q1)

    y0, y1, q0, q1 = jax.lax.fori_loop(0, MAX_ITER, body, (y0, y1, q0, q1))
    t_ref[...] = jnp.exp(y1)


def _sinkhorn(att2d, eb2d, stats, interpret=False):
    return pl.pallas_call(
        _sinkhorn_body,
        out_shape=jax.ShapeDtypeStruct((ER, LANE), jnp.float32),
        interpret=interpret,
    )(att2d, eb2d, stats)


def _gin(x_np, srcp, dstp, Wenc, benc, W1, b1, W2, b2, edge_att=None):
    h = x_np @ Wenc + benc
    nl = W1.shape[0]
    for l in range(nl):
        agg = _sc_agg(h, dstp, srcp, edge_att)
        z = h + agg
        z = jax.nn.relu(z @ W1[l] + b1[l])
        z = z @ W2[l] + b2[l]
        h = jax.nn.relu(z) if l < nl - 1 else z
    return h


@jax.jit
def kernel(x, edge_src, edge_dst, node_batch, W_enc, b_enc, enc_W1, enc_b1,
           enc_W2, enc_b2, Wa1, ba1, Wa2, ba2, W_cenc, b_cenc, clf_W1, clf_b1,
           clf_W2, clf_b2, W_out, b_out):
    edge_batch = node_batch[edge_src]
    deg = jnp.bincount(edge_batch, length=B)
    degf = deg.astype(jnp.float32)

    x_np = jnp.pad(x, ((0, NP - N), (0, 0)))
    srcp = jnp.pad(edge_src, (0, EPADE - E), constant_values=N)
    dstp = jnp.pad(edge_dst, (0, EPADE - E), constant_values=0)

    h = _gin(x_np, srcp, dstp, W_enc, b_enc, enc_W1, enc_b1, enc_W2, enc_b2)
    hs = h[edge_src]
    hd = h[edge_dst]
    att = (jax.nn.relu(jnp.concatenate([hs, hd], -1) @ Wa1 + ba1) @ Wa2 + ba2)[:, 0]

    gsum = jax.ops.segment_sum(att, edge_batch, num_segments=B)
    gsq = jax.ops.segment_sum(att * att, edge_batch, num_segments=B)
    gmin = jax.ops.segment_min(att, edge_batch, num_segments=B)
    stats = jnp.zeros((8, LANE), jnp.float32)
    stats = stats.at[0, :B].set(gsum).at[1, :B].set(gsq)
    stats = stats.at[2, :B].set(gmin).at[3, :B].set(degf)

    T2d = _sinkhorn(att.reshape(ER, LANE), edge_batch.reshape(ER, LANE), stats)
    T_edge = T2d.reshape(E)

    Tp = jnp.pad(T_edge, (0, EPADE - E))
    hc = _gin(x_np, srcp, dstp, W_cenc, b_cenc, clf_W1, clf_b1, clf_W2,
              clf_b2, edge_att=Tp)[:N]
    ncount = jnp.bincount(node_batch, length=B).astype(jnp.float32)
    hg = jax.ops.segment_sum(hc, node_batch, num_segments=B) / ncount[:, None]
    return hg @ W_out + b_out
